# Initial kernel scaffold; baseline (speedup 1.0000x reference)
import jax
import jax.numpy as jnp
from jax.experimental import pallas as pl


def kernel(x, edge_index, edge_attr, params):
    raise NotImplementedError("write your pallas kernel here")


# speedup vs baseline: 1.5084x; 1.5084x over previous
import functools

import jax
import jax.numpy as jnp
from jax import lax
from jax.experimental import pallas as pl
from jax.experimental.pallas import tpu as pltpu
from jax.experimental.pallas import tpu_sc as plsc

N = 10000
E = 320000
H = 128

NPAD = 10112
EPAD = 327680
NW = 32
EW = EPAD // NW
G = 256
CH = 1024
NT = EW // CH
RPT = NPAD // 16

_f32 = jnp.float32


def _ln(h, g, beta):
    mu = jnp.mean(h, axis=-1, keepdims=True)
    r = h - mu
    var = jnp.mean(r * r, axis=-1, keepdims=True)
    return r * lax.rsqrt(var + 1e-5) * g + beta


def _proj_body(x_ref, wp_ref, wqr_ref, bp_ref, bqr_ref, p_ref, qr_ref):
    x = x_ref[...]
    p_ref[...] = jnp.dot(x, wp_ref[...], preferred_element_type=_f32) + bp_ref[...]
    qr_ref[...] = jnp.dot(x, wqr_ref[...], preferred_element_type=_f32) + bqr_ref[...]


@functools.cache
def _proj_call():
    bn = 2000
    full = lambda *s: pl.BlockSpec(s, lambda i: (0,) * len(s))
    return pl.pallas_call(
        _proj_body,
        grid=(N // bn,),
        in_specs=[
            pl.BlockSpec((bn, 128), lambda i: (i, 0)),
            full(128, 128), full(128, 256), full(1, 128), full(1, 256),
        ],
        out_specs=[
            pl.BlockSpec((bn, 128), lambda i: (i, 0)),
            pl.BlockSpec((bn, 256), lambda i: (i, 0)),
        ],
        out_shape=[
            jax.ShapeDtypeStruct((N, 128), _f32),
            jax.ShapeDtypeStruct((N, 256), _f32),
        ],
    )


def _edge_body(gp_ref, gqr_ref, ea_ref, w1e_ref, g1_ref, be1_ref, w2_ref,
               b2_ref, v1e_ref, g2_ref, be2_ref, v2_ref, c2_ref,
               e_ref, m_ref):
    gqr = gqr_ref[...]
    h = gp_ref[...] + gqr[:, :128]
    h = h + jnp.dot(ea_ref[...], w1e_ref[...], preferred_element_type=_f32)
    h = _ln(jnp.maximum(h, 0.0), g1_ref[...], be1_ref[...])
    e = jnp.dot(h, w2_ref[...], preferred_element_type=_f32) + b2_ref[...]
    e_ref[...] = e
    h2 = gqr[:, 128:] + jnp.dot(e, v1e_ref[...], preferred_element_type=_f32)
    h2 = _ln(jnp.maximum(h2, 0.0), g2_ref[...], be2_ref[...])
    m_ref[...] = jnp.dot(h2, v2_ref[...], preferred_element_type=_f32) + c2_ref[...]


@functools.cache
def _edge_call(ef):
    be = 1024
    full = lambda *s: pl.BlockSpec(s, lambda i: (0,) * len(s))
    return pl.pallas_call(
        _edge_body,
        grid=(EPAD // be,),
        in_specs=[
            pl.BlockSpec((be, 128), lambda i: (i, 0)),
            pl.BlockSpec((be, 256), lambda i: (i, 0)),
            pl.BlockSpec((be, ef), lambda i: (i, 0)),
            full(ef, 128), full(1, 128), full(1, 128), full(128, 128),
            full(1, 128), full(128, 128), full(1, 128), full(1, 128),
            full(128, 128), full(1, 128),
        ],
        out_specs=[
            pl.BlockSpec((be, 128), lambda i: (i, 0)),
            pl.BlockSpec((be, 128), lambda i: (i, 0)),
        ],
        out_shape=[
            jax.ShapeDtypeStruct((EPAD, 128), _f32),
            jax.ShapeDtypeStruct((EPAD, 128), _f32),
        ],
    )


def _node_body(x_ref, s_ref, cnt_ref, u1x_ref, u1m_ref, d1_ref, g_ref,
               be_ref, u2_ref, d2_ref, o_ref):
    ssum = s_ref[0] + s_ref[1]
    csum = cnt_ref[0][:, :1] + cnt_ref[1][:, :1]
    mean = ssum / jnp.maximum(csum, 1.0)
    h = jnp.dot(x_ref[...], u1x_ref[...], preferred_element_type=_f32)
    h = h + jnp.dot(mean, u1m_ref[...], preferred_element_type=_f32) + d1_ref[...]
    h = _ln(jnp.maximum(h, 0.0), g_ref[...], be_ref[...])
    o_ref[...] = jnp.dot(h, u2_ref[...], preferred_element_type=_f32) + d2_ref[...]


@functools.cache
def _node_call():
    bn = 2000
    full = lambda *s: pl.BlockSpec(s, lambda i: (0,) * len(s))
    return pl.pallas_call(
        _node_body,
        grid=(N // bn,),
        in_specs=[
            pl.BlockSpec((bn, 128), lambda i: (i, 0)),
            pl.BlockSpec((2, bn, 128), lambda i: (0, i, 0)),
            pl.BlockSpec((2, bn, 128), lambda i: (0, i, 0)),
            full(128, 128), full(128, 128), full(1, 128), full(1, 128),
            full(1, 128), full(128, 128), full(1, 128),
        ],
        out_specs=pl.BlockSpec((bn, 128), lambda i: (i, 0)),
        out_shape=jax.ShapeDtypeStruct((N, 128), _f32),
    )


def _gather_body(p_hbm, qr_hbm, ridx_hbm, cidx_hbm, gp_hbm, gqr_hbm,
                 idxr_v, idxc_v, bufp_v, bufqr_v, sem):
    c = lax.axis_index("c")
    s = lax.axis_index("s")
    wid = s * 2 + c

    def step(t, carry):
        chunk = pl.multiple_of(wid * EW + t * CH, CH)
        r0 = pl.multiple_of(chunk // 128, 8)
        pltpu.sync_copy(ridx_hbm.at[pl.ds(r0, 8)], idxr_v)
        pltpu.sync_copy(cidx_hbm.at[pl.ds(r0, 8)], idxc_v)
        for k in range(CH // G):
            base = pl.multiple_of(chunk + k * G, G)
            cps = []
            for j in range(2):
                cps.append(pltpu.async_copy(
                    p_hbm.at[idxr_v.at[2 * k + j]],
                    bufp_v.at[pl.ds(j * 128, 128)], sem))
            for j in range(2):
                cps.append(pltpu.async_copy(
                    qr_hbm.at[idxc_v.at[2 * k + j]],
                    bufqr_v.at[pl.ds(j * 128, 128)], sem))
            for cp in cps:
                cp.wait()
            pltpu.sync_copy(bufp_v, gp_hbm.at[pl.ds(base, G)])
            pltpu.sync_copy(bufqr_v, gqr_hbm.at[pl.ds(base, G)])
        return carry

    lax.fori_loop(0, NT, step, 0)


@functools.cache
def _gather_call():
    mesh = plsc.VectorSubcoreMesh(core_axis_name="c", subcore_axis_name="s")
    return pl.kernel(
        _gather_body,
        out_type=(
            jax.ShapeDtypeStruct((EPAD, 128), _f32),
            jax.ShapeDtypeStruct((EPAD, 256), _f32),
        ),
        mesh=mesh,
        scratch_types=[
            pltpu.VMEM((8, 128), jnp.int32),
            pltpu.VMEM((8, 128), jnp.int32),
            pltpu.VMEM((G, 128), _f32),
            pltpu.VMEM((G, 256), _f32),
            pltpu.SemaphoreType.DMA,
        ],
    )


def _scatter_body(m_hbm, sidx_hbm, zeros_hbm, out_hbm, shared, idx_v, mbuf_v):
    c = lax.axis_index("c")
    s = lax.axis_index("s")
    wid = s * 2 + c
    pltpu.sync_copy(zeros_hbm.at[pl.ds(pl.multiple_of(s * RPT, 8), RPT)],
                    shared.at[pl.ds(pl.multiple_of(s * RPT, 8), RPT)])
    plsc.subcore_barrier()

    def step(t, carry):
        chunk = pl.multiple_of(wid * EW + t * CH, CH)
        pltpu.sync_copy(sidx_hbm.at[pl.ds(pl.multiple_of(chunk // 128, 8), 8)], idx_v)
        for k in range(CH // G):
            base = pl.multiple_of(chunk + k * G, G)
            pltpu.sync_copy(m_hbm.at[pl.ds(base, G)], mbuf_v)
            for j in range(2):
                pltpu.sync_copy(mbuf_v.at[pl.ds(j * 128, 128)],
                                shared.at[idx_v.at[2 * k + j]], add=True)
        return carry

    lax.fori_loop(0, NT, step, 0)
    plsc.subcore_barrier()
    pltpu.sync_copy(shared.at[pl.ds(pl.multiple_of(s * RPT, 8), RPT)],
                    out_hbm.at[c, pl.ds(pl.multiple_of(s * RPT, 8), RPT)])


@functools.cache
def _scatter_call():
    mesh = plsc.VectorSubcoreMesh(core_axis_name="c", subcore_axis_name="s")
    return pl.kernel(
        _scatter_body,
        out_type=jax.ShapeDtypeStruct((2, NPAD, 128), _f32),
        mesh=mesh,
        scratch_types=[
            pltpu.VMEM_SHARED((NPAD, 128), _f32),
            pltpu.VMEM((8, 128), jnp.int32),
            pltpu.VMEM((G, 128), _f32),
        ],
    )


def _counts_body(sidx_hbm, zeros_hbm, ones_hbm, out_hbm, shared, idx_v, ones_v):
    c = lax.axis_index("c")
    s = lax.axis_index("s")
    wid = s * 2 + c
    pltpu.sync_copy(zeros_hbm.at[pl.ds(pl.multiple_of(s * RPT, 8), RPT)],
                    shared.at[pl.ds(pl.multiple_of(s * RPT, 8), RPT)])
    pltpu.sync_copy(ones_hbm, ones_v)
    plsc.subcore_barrier()

    def step(t, carry):
        chunk = pl.multiple_of(wid * EW + t * CH, CH)
        pltpu.sync_copy(sidx_hbm.at[pl.ds(pl.multiple_of(chunk // 128, 8), 8)], idx_v)
        for j in range(8):
            pltpu.sync_copy(ones_v, shared.at[idx_v.at[j]], add=True)
        return carry

    lax.fori_loop(0, NT, step, 0)
    plsc.subcore_barrier()
    pltpu.sync_copy(shared.at[pl.ds(pl.multiple_of(s * RPT, 8), RPT)],
                    out_hbm.at[c, pl.ds(pl.multiple_of(s * RPT, 8), RPT)])


@functools.cache
def _counts_call():
    mesh = plsc.VectorSubcoreMesh(core_axis_name="c", subcore_axis_name="s")
    return pl.kernel(
        _counts_body,
        out_type=jax.ShapeDtypeStruct((2, NPAD, 128), _f32),
        mesh=mesh,
        scratch_types=[
            pltpu.VMEM_SHARED((NPAD, 128), _f32),
            pltpu.VMEM((8, 128), jnp.int32),
            pltpu.VMEM((128, 128), _f32),
        ],
    )


def kernel(x, edge_index, edge_attr, params):
    ridx = edge_index[0].astype(jnp.int32)
    cidx = edge_index[1].astype(jnp.int32)
    pad = EPAD - E
    ridx_g = jnp.concatenate([ridx, jnp.zeros((pad,), jnp.int32)]).reshape(EPAD // 128, 128)
    cidx_g = jnp.concatenate([cidx, jnp.zeros((pad,), jnp.int32)]).reshape(EPAD // 128, 128)
    sidx = jnp.concatenate([ridx, jnp.full((pad,), N, jnp.int32)]).reshape(EPAD // 128, 128)

    zeros_n = jnp.zeros((NPAD, 128), _f32)
    ones_c = jnp.ones((128, 128), _f32)

    cnt2 = _counts_call()(sidx, zeros_n, ones_c)

    ea = jnp.pad(edge_attr, ((0, pad), (0, 0)))
    for p in params:
        pe, p1, p2 = p["edge"], p["n1"], p["n2"]
        ef = ea.shape[1]
        wp = pe["W1"][:128]
        wqr = jnp.concatenate([pe["W1"][128:256], p1["W1"][:128]], axis=1)
        bp = pe["b1"].reshape(1, 128)
        bqr = jnp.concatenate([jnp.zeros((128,), _f32), p1["b1"]]).reshape(1, 256)
        pt, qrt = _proj_call()(x, wp, wqr, bp, bqr)
        gp, gqr = _gather_call()(pt, qrt, ridx_g, cidx_g)
        e, m = _edge_call(ef)(
            gp, gqr, ea, pe["W1"][256:],
            pe["g"].reshape(1, 128), pe["beta"].reshape(1, 128),
            pe["W2"], pe["b2"].reshape(1, 128),
            p1["W1"][128:],
            p1["g"].reshape(1, 128), p1["beta"].reshape(1, 128),
            p1["W2"], p1["b2"].reshape(1, 128),
        )
        s2 = _scatter_call()(m, sidx, zeros_n)
        x = _node_call()(
            x, s2, cnt2,
            p2["W1"][:128], p2["W1"][128:], p2["b1"].reshape(1, 128),
            p2["g"].reshape(1, 128), p2["beta"].reshape(1, 128),
            p2["W2"], p2["b2"].reshape(1, 128),
        )
        ea = e
    return x


# speedup vs baseline: 1.7574x; 1.1651x over previous
import functools

import jax
import jax.numpy as jnp
from jax import lax
from jax.experimental import pallas as pl
from jax.experimental.pallas import tpu as pltpu
from jax.experimental.pallas import tpu_sc as plsc

N = 10000
E = 320000
H = 128

NPAD = 10112
EPAD = 327680
NW = 32
EW = EPAD // NW
G = 256
CH = 1024
NT = EW // CH
RPT = NPAD // 16

_f32 = jnp.float32


def _ln(h, g, beta):
    mu = jnp.mean(h, axis=-1, keepdims=True)
    r = h - mu
    var = jnp.mean(r * r, axis=-1, keepdims=True)
    return r * lax.rsqrt(var + 1e-5) * g + beta


def _proj_body(x_ref, wp_ref, wqr_ref, bp_ref, bqr_ref, p_ref, qr_ref):
    x = x_ref[...]
    p_ref[...] = jnp.dot(x, wp_ref[...], preferred_element_type=_f32) + bp_ref[...]
    qr_ref[...] = jnp.dot(x, wqr_ref[...], preferred_element_type=_f32) + bqr_ref[...]


@functools.cache
def _proj_call():
    bn = 2000
    full = lambda *s: pl.BlockSpec(s, lambda i: (0,) * len(s))
    return pl.pallas_call(
        _proj_body,
        grid=(N // bn,),
        in_specs=[
            pl.BlockSpec((bn, 128), lambda i: (i, 0)),
            full(128, 128), full(128, 256), full(1, 128), full(1, 256),
        ],
        out_specs=[
            pl.BlockSpec((bn, 128), lambda i: (i, 0)),
            pl.BlockSpec((bn, 256), lambda i: (i, 0)),
        ],
        out_shape=[
            jax.ShapeDtypeStruct((N, 128), _f32),
            jax.ShapeDtypeStruct((N, 256), _f32),
        ],
    )


def _edge_body(gp_ref, gqr_ref, ea_ref, w1e_ref, g1_ref, be1_ref, w2_ref,
               b2_ref, v1e_ref, g2_ref, be2_ref, v2_ref, c2_ref,
               e_ref, m_ref):
    gqr = gqr_ref[...]
    h = gp_ref[...] + gqr[:, :128]
    h = h + jnp.dot(ea_ref[...], w1e_ref[...], preferred_element_type=_f32)
    h = _ln(jnp.maximum(h, 0.0), g1_ref[...], be1_ref[...])
    e = jnp.dot(h, w2_ref[...], preferred_element_type=_f32) + b2_ref[...]
    e_ref[...] = e
    h2 = gqr[:, 128:] + jnp.dot(e, v1e_ref[...], preferred_element_type=_f32)
    h2 = _ln(jnp.maximum(h2, 0.0), g2_ref[...], be2_ref[...])
    m_ref[...] = jnp.dot(h2, v2_ref[...], preferred_element_type=_f32) + c2_ref[...]


@functools.cache
def _edge_call(ef):
    be = 1024
    full = lambda *s: pl.BlockSpec(s, lambda i: (0,) * len(s))
    return pl.pallas_call(
        _edge_body,
        grid=(EPAD // be,),
        in_specs=[
            pl.BlockSpec((be, 128), lambda i: (i, 0)),
            pl.BlockSpec((be, 256), lambda i: (i, 0)),
            pl.BlockSpec((be, ef), lambda i: (i, 0)),
            full(ef, 128), full(1, 128), full(1, 128), full(128, 128),
            full(1, 128), full(128, 128), full(1, 128), full(1, 128),
            full(128, 128), full(1, 128),
        ],
        out_specs=[
            pl.BlockSpec((be, 128), lambda i: (i, 0)),
            pl.BlockSpec((be, 128), lambda i: (i, 0)),
        ],
        out_shape=[
            jax.ShapeDtypeStruct((EPAD, 128), _f32),
            jax.ShapeDtypeStruct((EPAD, 128), _f32),
        ],
    )


def _node_body(x_ref, s_ref, cnt_ref, u1x_ref, u1m_ref, d1_ref, g_ref,
               be_ref, u2_ref, d2_ref, o_ref):
    ssum = s_ref[0] + s_ref[1]
    csum = cnt_ref[0][:, :1] + cnt_ref[1][:, :1]
    mean = ssum / jnp.maximum(csum, 1.0)
    h = jnp.dot(x_ref[...], u1x_ref[...], preferred_element_type=_f32)
    h = h + jnp.dot(mean, u1m_ref[...], preferred_element_type=_f32) + d1_ref[...]
    h = _ln(jnp.maximum(h, 0.0), g_ref[...], be_ref[...])
    o_ref[...] = jnp.dot(h, u2_ref[...], preferred_element_type=_f32) + d2_ref[...]


@functools.cache
def _node_call():
    bn = 2000
    full = lambda *s: pl.BlockSpec(s, lambda i: (0,) * len(s))
    return pl.pallas_call(
        _node_body,
        grid=(N // bn,),
        in_specs=[
            pl.BlockSpec((bn, 128), lambda i: (i, 0)),
            pl.BlockSpec((2, bn, 128), lambda i: (0, i, 0)),
            pl.BlockSpec((2, bn, 128), lambda i: (0, i, 0)),
            full(128, 128), full(128, 128), full(1, 128), full(1, 128),
            full(1, 128), full(128, 128), full(1, 128),
        ],
        out_specs=pl.BlockSpec((bn, 128), lambda i: (i, 0)),
        out_shape=jax.ShapeDtypeStruct((N, 128), _f32),
    )


def _gather_body(p_hbm, qr_hbm, ridx_hbm, cidx_hbm, gp_hbm, gqr_hbm,
                 idxr_v, idxc_v, bufp_v, bufqr_v, gsem0, gsem1, wsem0, wsem1):
    c = lax.axis_index("c")
    s = lax.axis_index("s")
    wid = s * 2 + c
    gsems = (gsem0, gsem1)
    wsems = (wsem0, wsem1)

    def step(t, carry):
        chunk = pl.multiple_of(wid * EW + t * CH, CH)
        r0 = pl.multiple_of(chunk // 128, 8)
        pltpu.sync_copy(ridx_hbm.at[pl.ds(r0, 8)], idxr_v)
        pltpu.sync_copy(cidx_hbm.at[pl.ds(r0, 8)], idxc_v)
        nk = CH // 128
        gcps = [None] * nk
        wcps = [None] * nk
        for k in range(nk):
            par = k & 1
            if k >= 2:
                for cp in wcps[k - 2]:
                    cp.wait()
            gcps[k] = [
                pltpu.async_copy(p_hbm.at[idxr_v.at[k]],
                                 bufp_v.at[par], gsems[par]),
                pltpu.async_copy(qr_hbm.at[idxc_v.at[k]],
                                 bufqr_v.at[par], gsems[par]),
            ]
            if k >= 1:
                pv = (k - 1) & 1
                base = pl.multiple_of(chunk + (k - 1) * 128, 128)
                for cp in gcps[k - 1]:
                    cp.wait()
                wcps[k - 1] = [
                    pltpu.async_copy(bufp_v.at[pv],
                                     gp_hbm.at[pl.ds(base, 128)], wsems[pv]),
                    pltpu.async_copy(bufqr_v.at[pv],
                                     gqr_hbm.at[pl.ds(base, 128)], wsems[pv]),
                ]
        last = nk - 1
        for cp in gcps[last]:
            cp.wait()
        base = pl.multiple_of(chunk + last * 128, 128)
        wcps[last] = [
            pltpu.async_copy(bufp_v.at[last & 1],
                             gp_hbm.at[pl.ds(base, 128)], wsems[last & 1]),
            pltpu.async_copy(bufqr_v.at[last & 1],
                             gqr_hbm.at[pl.ds(base, 128)], wsems[last & 1]),
        ]
        for k in (nk - 2, nk - 1):
            for cp in wcps[k]:
                cp.wait()
        return carry

    lax.fori_loop(0, NT, step, 0)


@functools.cache
def _gather_call():
    mesh = plsc.VectorSubcoreMesh(core_axis_name="c", subcore_axis_name="s")
    return pl.kernel(
        _gather_body,
        out_type=(
            jax.ShapeDtypeStruct((EPAD, 128), _f32),
            jax.ShapeDtypeStruct((EPAD, 256), _f32),
        ),
        mesh=mesh,
        scratch_types=[
            pltpu.VMEM((8, 128), jnp.int32),
            pltpu.VMEM((8, 128), jnp.int32),
            pltpu.VMEM((2, 128, 128), _f32),
            pltpu.VMEM((2, 128, 256), _f32),
            pltpu.SemaphoreType.DMA,
            pltpu.SemaphoreType.DMA,
            pltpu.SemaphoreType.DMA,
            pltpu.SemaphoreType.DMA,
        ],
    )


def _scatter_body(m_hbm, sidx_hbm, zeros_hbm, out_hbm, shared, idx_v, mbuf_v):
    c = lax.axis_index("c")
    s = lax.axis_index("s")
    wid = s * 2 + c
    pltpu.sync_copy(zeros_hbm.at[pl.ds(pl.multiple_of(s * RPT, 8), RPT)],
                    shared.at[pl.ds(pl.multiple_of(s * RPT, 8), RPT)])
    plsc.subcore_barrier()

    def step(t, carry):
        chunk = pl.multiple_of(wid * EW + t * CH, CH)
        pltpu.sync_copy(sidx_hbm.at[pl.ds(pl.multiple_of(chunk // 128, 8), 8)], idx_v)
        for k in range(CH // G):
            base = pl.multiple_of(chunk + k * G, G)
            pltpu.sync_copy(m_hbm.at[pl.ds(base, G)], mbuf_v)
            for j in range(2):
                pltpu.sync_copy(mbuf_v.at[pl.ds(j * 128, 128)],
                                shared.at[idx_v.at[2 * k + j]], add=True)
        return carry

    lax.fori_loop(0, NT, step, 0)
    plsc.subcore_barrier()
    pltpu.sync_copy(shared.at[pl.ds(pl.multiple_of(s * RPT, 8), RPT)],
                    out_hbm.at[c, pl.ds(pl.multiple_of(s * RPT, 8), RPT)])


@functools.cache
def _scatter_call():
    mesh = plsc.VectorSubcoreMesh(core_axis_name="c", subcore_axis_name="s")
    return pl.kernel(
        _scatter_body,
        out_type=jax.ShapeDtypeStruct((2, NPAD, 128), _f32),
        mesh=mesh,
        scratch_types=[
            pltpu.VMEM_SHARED((NPAD, 128), _f32),
            pltpu.VMEM((8, 128), jnp.int32),
            pltpu.VMEM((G, 128), _f32),
        ],
    )


def _counts_body(sidx_hbm, zeros_hbm, ones_hbm, out_hbm, shared, idx_v, ones_v):
    c = lax.axis_index("c")
    s = lax.axis_index("s")
    wid = s * 2 + c
    pltpu.sync_copy(zeros_hbm.at[pl.ds(pl.multiple_of(s * RPT, 8), RPT)],
                    shared.at[pl.ds(pl.multiple_of(s * RPT, 8), RPT)])
    pltpu.sync_copy(ones_hbm, ones_v)
    plsc.subcore_barrier()

    def step(t, carry):
        chunk = pl.multiple_of(wid * EW + t * CH, CH)
        pltpu.sync_copy(sidx_hbm.at[pl.ds(pl.multiple_of(chunk // 128, 8), 8)], idx_v)
        for j in range(8):
            pltpu.sync_copy(ones_v, shared.at[idx_v.at[j]], add=True)
        return carry

    lax.fori_loop(0, NT, step, 0)
    plsc.subcore_barrier()
    pltpu.sync_copy(shared.at[pl.ds(pl.multiple_of(s * RPT, 8), RPT)],
                    out_hbm.at[c, pl.ds(pl.multiple_of(s * RPT, 8), RPT)])


@functools.cache
def _counts_call():
    mesh = plsc.VectorSubcoreMesh(core_axis_name="c", subcore_axis_name="s")
    return pl.kernel(
        _counts_body,
        out_type=jax.ShapeDtypeStruct((2, NPAD, 128), _f32),
        mesh=mesh,
        scratch_types=[
            pltpu.VMEM_SHARED((NPAD, 128), _f32),
            pltpu.VMEM((8, 128), jnp.int32),
            pltpu.VMEM((128, 128), _f32),
        ],
    )


def kernel(x, edge_index, edge_attr, params):
    ridx = edge_index[0].astype(jnp.int32)
    cidx = edge_index[1].astype(jnp.int32)
    pad = EPAD - E
    ridx_g = jnp.concatenate([ridx, jnp.zeros((pad,), jnp.int32)]).reshape(EPAD // 128, 128)
    cidx_g = jnp.concatenate([cidx, jnp.zeros((pad,), jnp.int32)]).reshape(EPAD // 128, 128)
    sidx = jnp.concatenate([ridx, jnp.full((pad,), N, jnp.int32)]).reshape(EPAD // 128, 128)

    zeros_n = jnp.zeros((NPAD, 128), _f32)
    ones_c = jnp.ones((128, 128), _f32)

    cnt2 = _counts_call()(sidx, zeros_n, ones_c)

    ea = jnp.pad(edge_attr, ((0, pad), (0, 0)))
    for p in params:
        pe, p1, p2 = p["edge"], p["n1"], p["n2"]
        ef = ea.shape[1]
        wp = pe["W1"][:128]
        wqr = jnp.concatenate([pe["W1"][128:256], p1["W1"][:128]], axis=1)
        bp = pe["b1"].reshape(1, 128)
        bqr = jnp.concatenate([jnp.zeros((128,), _f32), p1["b1"]]).reshape(1, 256)
        pt, qrt = _proj_call()(x, wp, wqr, bp, bqr)
        gp, gqr = _gather_call()(pt, qrt, ridx_g, cidx_g)
        e, m = _edge_call(ef)(
            gp, gqr, ea, pe["W1"][256:],
            pe["g"].reshape(1, 128), pe["beta"].reshape(1, 128),
            pe["W2"], pe["b2"].reshape(1, 128),
            p1["W1"][128:],
            p1["g"].reshape(1, 128), p1["beta"].reshape(1, 128),
            p1["W2"], p1["b2"].reshape(1, 128),
        )
        s2 = _scatter_call()(m, sidx, zeros_n)
        x = _node_call()(
            x, s2, cnt2,
            p2["W1"][:128], p2["W1"][128:], p2["b1"].reshape(1, 128),
            p2["g"].reshape(1, 128), p2["beta"].reshape(1, 128),
            p2["W2"], p2["b2"].reshape(1, 128),
        )
        ea = e
    return x
